# baseline (speedup 1.0000x reference)
import functools

import jax
import jax.numpy as jnp
from jax import lax
from jax.experimental import pallas as pl
from jax.experimental.pallas import tpu as pltpu
from jax.experimental.pallas import tpu_sc as plsc

N = 10000
E = 320000
F_IN = 128
HID = 64
NC = 16
G = 64
IMG = 50176

NCORES = 2
NSUB = 16
NW = NCORES * NSUB

CHUNK = 128
NPAD = 10240
RPT = NPAD // NSUB
NCOPY = RPT // CHUNK
NITER = 80
EPW = NITER * CHUNK
EPAD = EPW * NW
NITER_T = EPAD // (NSUB * CHUNK)

_HIGH = jax.lax.Precision.HIGHEST


def _dot(a, b):
    return jnp.dot(a, b, precision=_HIGH, preferred_element_type=jnp.float32)


DW = 16


def _sc_hist(dst2):
    mesh = plsc.VectorSubcoreMesh(core_axis_name="c", subcore_axis_name="s")

    @functools.partial(
        pl.kernel,
        out_type=jax.ShapeDtypeStruct((NCORES, NPAD, DW), jnp.float32),
        mesh=mesh,
        compiler_params=pltpu.CompilerParams(use_tc_tiling_on_sc=False),
        scratch_types=[
            pltpu.VMEM((NITER, CHUNK), jnp.int32),
            pltpu.VMEM((CHUNK, DW), jnp.float32),
            pltpu.VMEM_SHARED((NPAD, DW), jnp.float32),
        ],
    )
    def k(dst_hbm, out_hbm, dsta, buf, acc):
        c = lax.axis_index("c")
        s = lax.axis_index("s")
        w = c * NSUB + s

        def fill(val):
            def body(r, _):
                buf[r, pl.ds(0, DW)] = jnp.full((DW,), val, jnp.float32)
                return 0
            lax.fori_loop(0, CHUNK, body, 0)

        fill(0.0)
        for j in range(NCOPY):
            pltpu.sync_copy(buf, acc.at[pl.ds(s * RPT + j * CHUNK, CHUNK)])
        fill(1.0)
        pltpu.sync_copy(dst_hbm.at[pl.ds(w * NITER, NITER)], dsta)
        plsc.subcore_barrier()

        def step(g, _):
            pltpu.sync_copy(buf, acc.at[dsta.at[g]], add=True)
            return 0

        lax.fori_loop(0, NITER, step, 0)
        plsc.subcore_barrier()

        for j in range(NCOPY):
            r0 = s * RPT + j * CHUNK
            pltpu.sync_copy(acc.at[pl.ds(r0, CHUNK)], buf)
            pltpu.sync_copy(buf, out_hbm.at[c, pl.ds(r0, CHUNK)])

    return k(dst2)


def _sc_aggregate(src3, dst3, tab_split, Dh, stage_tab=False):
    NBUF = 4
    LEAD = 2
    mesh = plsc.VectorSubcoreMesh(core_axis_name="c", subcore_axis_name="s")

    @functools.partial(
        pl.kernel,
        out_type=jax.ShapeDtypeStruct((NCORES, NPAD, Dh), jnp.float32),
        mesh=mesh,
        compiler_params=pltpu.CompilerParams(use_tc_tiling_on_sc=False),
        scratch_types=[
            pltpu.VMEM((NITER_T, CHUNK), jnp.int32),
            pltpu.VMEM((NITER_T, CHUNK), jnp.int32),
            [pltpu.VMEM((CHUNK, Dh), jnp.float32) for _ in range(NBUF)],
            pltpu.VMEM_SHARED((NPAD, Dh), jnp.float32),
        ] + ([pltpu.VMEM_SHARED((NPAD, Dh), jnp.float32)] if stage_tab else []) + [
            [pltpu.SemaphoreType.DMA for _ in range(NBUF)],
            [pltpu.SemaphoreType.DMA for _ in range(NBUF)],
        ],
    )
    def k(src_hbm, dst_hbm, tab_hbm, out_hbm, srca, dsta, rows, acc, *rest):
        if stage_tab:
            tabs, gsem, ssem = rest
        else:
            tabs = None
            gsem, ssem = rest
        c = lax.axis_index("c")
        s = lax.axis_index("s")
        if stage_tab:
            for j in range(NCOPY):
                r0 = s * RPT + j * CHUNK
                pltpu.sync_copy(tab_hbm.at[c, pl.ds(r0, CHUNK)], rows[0])
                pltpu.sync_copy(rows[0], tabs.at[pl.ds(r0, CHUNK)])
            tab = tabs
        else:
            tab = tab_hbm.at[c]

        def zrow(r, _):
            for j in range(Dh // 16):
                rows[0][r, pl.ds(j * 16, 16)] = jnp.zeros((16,), jnp.float32)
            return 0

        lax.fori_loop(0, CHUNK, zrow, 0)
        for j in range(NCOPY):
            pltpu.sync_copy(rows[0], acc.at[pl.ds(s * RPT + j * CHUNK, CHUNK)])
        pltpu.sync_copy(src_hbm.at[pl.ds(s * NITER_T, NITER_T)], srca)
        pltpu.sync_copy(dst_hbm.at[pl.ds(s * NITER_T, NITER_T)], dsta)
        plsc.subcore_barrier()

        for b in range(LEAD):
            pltpu.async_copy(tab.at[srca.at[b]], rows[b], gsem[b])

        def quad(q, _):
            for b in range(NBUF):
                g = NBUF * q + b
                bg = (b + LEAD) % NBUF
                g2 = g + LEAD

                @pl.when(jnp.logical_and(g2 >= NBUF, g2 < NITER_T))
                def _():
                    pltpu.make_async_copy(rows[bg], acc.at[dsta.at[g2 - NBUF]],
                                          ssem[bg]).wait()

                @pl.when(g2 < NITER_T)
                def _():
                    pltpu.async_copy(tab.at[srca.at[g2]], rows[bg], gsem[bg])

                pltpu.make_async_copy(tab.at[srca.at[g]], rows[b], gsem[b]).wait()
                pltpu.async_copy(rows[b], acc.at[dsta.at[g]], ssem[b], add=True)
            return 0

        lax.fori_loop(0, NITER_T // NBUF, quad, 0)
        for b in range(NBUF):
            g = NITER_T - NBUF + b
            pltpu.make_async_copy(rows[b], acc.at[dsta.at[g]], ssem[b]).wait()
        plsc.subcore_barrier()

        for j in range(NCOPY):
            r0 = s * RPT + j * CHUNK
            pltpu.sync_copy(acc.at[pl.ds(r0, CHUNK)], rows[0])
            pltpu.sync_copy(rows[0], out_hbm.at[c, pl.ds(r0, CHUNK)])

    return k(src3, dst3, tab_split)


RB = 2048
KB = 1024


KA = 25


def _imga_body(img_ref, wm0_ref, out_ref, acc_ref):
    kstep = pl.program_id(0)

    @pl.when(kstep == 0)
    def _():
        acc_ref[...] = jnp.zeros_like(acc_ref)

    acc_ref[...] += _dot(img_ref[...], wm0_ref[...])

    @pl.when(kstep == KA - 1)
    def _():
        out_ref[...] = acc_ref[...]


def _img_mlp_a(img, Wm0):
    return pl.pallas_call(
        _imga_body,
        grid=(KA,),
        in_specs=[
            pl.BlockSpec((G, KB), lambda k: (0, k)),
            pl.BlockSpec((KB, 1024), lambda k: (k, 0)),
        ],
        out_specs=pl.BlockSpec((G, 1024), lambda k: (0, 0)),
        out_shape=jax.ShapeDtypeStruct((G, 1024), jnp.float32),
        scratch_shapes=[pltpu.VMEM((G, 1024), jnp.float32)],
    )(img, Wm0)


def _imgb_body(img_ref, wm0_ref, bm0_ref, wm1_ref, bm1_ref, part_ref,
               out_ref, acc_ref):
    kstep = pl.program_id(0)

    @pl.when(kstep == 0)
    def _():
        acc_ref[...] = part_ref[...]

    acc_ref[...] += _dot(img_ref[...], wm0_ref[...])

    @pl.when(kstep == IMG // KB - KA - 1)
    def _():
        y = acc_ref[...] + bm0_ref[...][None, :]
        out_ref[...] = _dot(y, wm1_ref[...]) + bm1_ref[...][None, :]


def _img_mlp_b(img, Wm0, bm0, Wm1, bm1, part):
    return pl.pallas_call(
        _imgb_body,
        grid=(IMG // KB - KA,),
        in_specs=[
            pl.BlockSpec((G, KB), lambda k: (0, k + KA)),
            pl.BlockSpec((KB, 1024), lambda k: (k + KA, 0)),
            pl.BlockSpec((1024,), lambda k: (0,)),
            pl.BlockSpec((1024, HID), lambda k: (0, 0)),
            pl.BlockSpec((HID,), lambda k: (0,)),
            pl.BlockSpec((G, 1024), lambda k: (0, 0)),
        ],
        out_specs=pl.BlockSpec((G, HID), lambda k: (0, 0)),
        out_shape=jax.ShapeDtypeStruct((G, HID), jnp.float32),
        scratch_shapes=[pltpu.VMEM((G, 1024), jnp.float32)],
    )(img, Wm0, bm0, Wm1, bm1, part)


def _xw_body(x_ref, w1_ref, xw_ref):
    xw_ref[...] = _dot(x_ref[...], w1_ref[...])


def _xw(x_pad, W1):
    return pl.pallas_call(
        _xw_body,
        grid=(NPAD // RB,),
        in_specs=[
            pl.BlockSpec((RB, F_IN), lambda i: (i, 0)),
            pl.BlockSpec((F_IN, F_IN), lambda i: (0, 0)),
        ],
        out_specs=pl.BlockSpec((RB, F_IN), lambda i: (i, 0)),
        out_shape=jax.ShapeDtypeStruct((NPAD, F_IN), jnp.float32),
    )(x_pad, W1)


QW = F_IN // 4


def _prescale_body(deg_ref, xw_ref, dinv_ref, hs1a_ref, hs1b_ref):
    deg = deg_ref[0, :, 0] + deg_ref[1, :, 0] + 1.0
    dinv = lax.rsqrt(jnp.maximum(deg, 1.0))
    dinv_ref[...] = dinv
    hs = xw_ref[...] * dinv[:, None]
    hs1a_ref[0] = hs[:, 0 * QW:1 * QW]
    hs1a_ref[1] = hs[:, 1 * QW:2 * QW]
    hs1b_ref[0] = hs[:, 2 * QW:3 * QW]
    hs1b_ref[1] = hs[:, 3 * QW:4 * QW]


def _prescale(deg_part, xw):
    return pl.pallas_call(
        _prescale_body,
        grid=(NPAD // RB,),
        in_specs=[
            pl.BlockSpec((NCORES, RB, DW), lambda i: (0, i, 0)),
            pl.BlockSpec((RB, F_IN), lambda i: (i, 0)),
        ],
        out_specs=[
            pl.BlockSpec((RB,), lambda i: (i,)),
            pl.BlockSpec((NCORES, RB, QW), lambda i: (0, i, 0)),
            pl.BlockSpec((NCORES, RB, QW), lambda i: (0, i, 0)),
        ],
        out_shape=[
            jax.ShapeDtypeStruct((NPAD,), jnp.float32),
            jax.ShapeDtypeStruct((NCORES, NPAD, QW), jnp.float32),
            jax.ShapeDtypeStruct((NCORES, NPAD, QW), jnp.float32),
        ],
    )(deg_part, xw)


def _mid_body(agga_ref, aggb_ref, hs1a_ref, hs1b_ref, dinv_ref, b1_ref,
              w2_ref, hs2_ref):
    dinv = dinv_ref[...]
    tot = (jnp.concatenate([agga_ref[0], agga_ref[1],
                            aggb_ref[0], aggb_ref[1]], axis=1)
           + jnp.concatenate([hs1a_ref[0], hs1a_ref[1],
                              hs1b_ref[0], hs1b_ref[1]], axis=1))
    h1 = jnp.maximum(tot * dinv[:, None] + b1_ref[...][None, :], 0.0)
    hs2 = _dot(h1, w2_ref[...]) * dinv[:, None]
    hs2_ref[0] = hs2[:, :HID // 2]
    hs2_ref[1] = hs2[:, HID // 2:]


def _mid_layer(agg1a, agg1b, hs1a, hs1b, dinv, b1, W2):
    return pl.pallas_call(
        _mid_body,
        grid=(NPAD // RB,),
        in_specs=[
            pl.BlockSpec((NCORES, RB, QW), lambda i: (0, i, 0)),
            pl.BlockSpec((NCORES, RB, QW), lambda i: (0, i, 0)),
            pl.BlockSpec((NCORES, RB, QW), lambda i: (0, i, 0)),
            pl.BlockSpec((NCORES, RB, QW), lambda i: (0, i, 0)),
            pl.BlockSpec((RB,), lambda i: (i,)),
            pl.BlockSpec((F_IN,), lambda i: (0,)),
            pl.BlockSpec((F_IN, HID), lambda i: (0, 0)),
        ],
        out_specs=pl.BlockSpec((NCORES, RB, HID // 2), lambda i: (0, i, 0)),
        out_shape=jax.ShapeDtypeStruct((NCORES, NPAD, HID // 2), jnp.float32),
    )(agg1a, agg1b, hs1a, hs1b, dinv, b1, W2)


def _final_body(agg_ref, hs2_ref, dinv_ref, b2_ref, batch_ref, x0_ref,
                wmx_ref, bmx_ref, wfc_ref, bfc_ref, out_ref, pool_ref):
    i = pl.program_id(0)

    @pl.when(i == 0)
    def _():
        pool_ref[...] = jnp.zeros_like(pool_ref)

    dinv = dinv_ref[...]
    tot = (jnp.concatenate([agg_ref[0], agg_ref[1]], axis=1)
           + jnp.concatenate([hs2_ref[0], hs2_ref[1]], axis=1))
    h2 = jnp.maximum(tot * dinv[:, None] + b2_ref[...][None, :], 0.0)
    gid = lax.broadcasted_iota(jnp.int32, (G, RB), 0)
    seg = (batch_ref[...][None, :] == gid).astype(jnp.float32)
    pool_ref[...] += _dot(seg, h2)

    @pl.when(i == NPAD // RB - 1)
    def _():
        xg = _dot(pool_ref[...], wmx_ref[...]) + bmx_ref[...][None, :]
        xt = jnp.concatenate([x0_ref[...], xg], axis=1)
        logits = _dot(xt, wfc_ref[...]) + bfc_ref[...][None, :]
        m = jnp.max(logits, axis=1, keepdims=True)
        lse = m + jnp.log(jnp.sum(jnp.exp(logits - m), axis=1, keepdims=True))
        out_ref[...] = logits - lse


def _final(agg2, hs2, dinv, b2, batch_pad, x0, Wmx, bmx, Wfc, bfc):
    return pl.pallas_call(
        _final_body,
        grid=(NPAD // RB,),
        in_specs=[
            pl.BlockSpec((NCORES, RB, HID // 2), lambda i: (0, i, 0)),
            pl.BlockSpec((NCORES, RB, HID // 2), lambda i: (0, i, 0)),
            pl.BlockSpec((RB,), lambda i: (i,)),
            pl.BlockSpec((HID,), lambda i: (0,)),
            pl.BlockSpec((RB,), lambda i: (i,)),
            pl.BlockSpec((G, HID), lambda i: (0, 0)),
            pl.BlockSpec((HID, HID), lambda i: (0, 0)),
            pl.BlockSpec((HID,), lambda i: (0,)),
            pl.BlockSpec((2 * HID, NC), lambda i: (0, 0)),
            pl.BlockSpec((NC,), lambda i: (0,)),
        ],
        out_specs=pl.BlockSpec((G, NC), lambda i: (0, 0)),
        out_shape=jax.ShapeDtypeStruct((G, NC), jnp.float32),
        scratch_shapes=[pltpu.VMEM((G, HID), jnp.float32)],
    )(agg2, hs2, dinv, b2, batch_pad, x0, Wmx, bmx, Wfc, bfc)


def kernel(x, edge_index, img_features, batch, W1, b1, W2, b2,
           Wm0, bm0, Wm1, bm1, Wmx, bmx, Wfc, bfc):
    src = edge_index[0].astype(jnp.int32)
    dst = edge_index[1].astype(jnp.int32)
    npd = EPAD - E
    src_pad = jnp.concatenate([src, jnp.full((npd,), N, jnp.int32)])
    dst_pad = jnp.concatenate(
        [dst, N + (jnp.arange(npd, dtype=jnp.int32) % (NPAD - N))])
    src2 = src_pad.reshape(EPAD // CHUNK, CHUNK)
    dst2 = dst_pad.reshape(EPAD // CHUNK, CHUNK)
    x_pad = jnp.pad(x, ((0, NPAD - N), (0, 0)))
    batch_pad = jnp.concatenate(
        [batch.astype(jnp.int32), jnp.full((NPAD - N,), G, jnp.int32)])
    part = _img_mlp_a(img_features, Wm0)

    xw = _xw(x_pad, W1)
    deg_part = _sc_hist(dst2)
    dinv, hs1a, hs1b = _prescale(deg_part, xw)
    agg1a = _sc_aggregate(src2, dst2, hs1a, QW, stage_tab=True)
    agg1b = _sc_aggregate(src2, dst2, hs1b, QW, stage_tab=True)
    hs2 = _mid_layer(agg1a, agg1b, hs1a, hs1b, dinv, b1, W2)
    hs2, part = lax.optimization_barrier((hs2, part))
    agg2 = _sc_aggregate(src2, dst2, hs2, HID // 2, stage_tab=True)
    x0 = _img_mlp_b(img_features, Wm0, bm0, Wm1, bm1, part)
    return _final(agg2, hs2, dinv, b2, batch_pad, x0, Wmx, bmx, Wfc, bfc)


# speedup vs baseline: 1.0063x; 1.0063x over previous
import functools

import jax
import jax.numpy as jnp
from jax import lax
from jax.experimental import pallas as pl
from jax.experimental.pallas import tpu as pltpu
from jax.experimental.pallas import tpu_sc as plsc

N = 10000
E = 320000
F_IN = 128
HID = 64
NC = 16
G = 64
IMG = 50176

NCORES = 2
NSUB = 16
NW = NCORES * NSUB

CHUNK = 128
NPAD = 10240
RPT = NPAD // NSUB
NCOPY = RPT // CHUNK
NITER = 80
EPW = NITER * CHUNK
EPAD = EPW * NW
NITER_T = EPAD // (NSUB * CHUNK)

_HIGH = jax.lax.Precision.HIGHEST


def _dot(a, b):
    return jnp.dot(a, b, precision=_HIGH, preferred_element_type=jnp.float32)


DW = 16


def _sc_hist(dst2):
    mesh = plsc.VectorSubcoreMesh(core_axis_name="c", subcore_axis_name="s")

    @functools.partial(
        pl.kernel,
        out_type=jax.ShapeDtypeStruct((NCORES, NPAD, DW), jnp.float32),
        mesh=mesh,
        compiler_params=pltpu.CompilerParams(use_tc_tiling_on_sc=False),
        scratch_types=[
            pltpu.VMEM((NITER, CHUNK), jnp.int32),
            pltpu.VMEM((CHUNK, DW), jnp.float32),
            pltpu.VMEM_SHARED((NPAD, DW), jnp.float32),
        ],
    )
    def k(dst_hbm, out_hbm, dsta, buf, acc):
        c = lax.axis_index("c")
        s = lax.axis_index("s")
        w = c * NSUB + s

        def fill(val):
            def body(r, _):
                buf[r, pl.ds(0, DW)] = jnp.full((DW,), val, jnp.float32)
                return 0
            lax.fori_loop(0, CHUNK, body, 0)

        fill(0.0)
        for j in range(NCOPY):
            pltpu.sync_copy(buf, acc.at[pl.ds(s * RPT + j * CHUNK, CHUNK)])
        fill(1.0)
        pltpu.sync_copy(dst_hbm.at[pl.ds(w * NITER, NITER)], dsta)
        plsc.subcore_barrier()

        def step(g, _):
            pltpu.sync_copy(buf, acc.at[dsta.at[g]], add=True)
            return 0

        lax.fori_loop(0, NITER, step, 0)
        plsc.subcore_barrier()

        for j in range(NCOPY):
            r0 = s * RPT + j * CHUNK
            pltpu.sync_copy(acc.at[pl.ds(r0, CHUNK)], buf)
            pltpu.sync_copy(buf, out_hbm.at[c, pl.ds(r0, CHUNK)])

    return k(dst2)


def _sc_aggregate(src3, dst3, tab_split, Dh, stage_tab=False):
    NBUF = 4
    LEAD = 2
    mesh = plsc.VectorSubcoreMesh(core_axis_name="c", subcore_axis_name="s")

    @functools.partial(
        pl.kernel,
        out_type=jax.ShapeDtypeStruct((NCORES, NPAD, Dh), jnp.float32),
        mesh=mesh,
        compiler_params=pltpu.CompilerParams(use_tc_tiling_on_sc=False),
        scratch_types=[
            pltpu.VMEM((NITER_T, CHUNK), jnp.int32),
            pltpu.VMEM((NITER_T, CHUNK), jnp.int32),
            [pltpu.VMEM((CHUNK, Dh), jnp.float32) for _ in range(NBUF)],
            pltpu.VMEM_SHARED((NPAD, Dh), jnp.float32),
        ] + ([pltpu.VMEM_SHARED((NPAD, Dh), jnp.float32)] if stage_tab else []) + [
            [pltpu.SemaphoreType.DMA for _ in range(NBUF)],
            [pltpu.SemaphoreType.DMA for _ in range(NBUF)],
        ],
    )
    def k(src_hbm, dst_hbm, tab_hbm, out_hbm, srca, dsta, rows, acc, *rest):
        if stage_tab:
            tabs, gsem, ssem = rest
        else:
            tabs = None
            gsem, ssem = rest
        c = lax.axis_index("c")
        s = lax.axis_index("s")
        if stage_tab:
            for j in range(NCOPY):
                r0 = s * RPT + j * CHUNK
                pltpu.sync_copy(tab_hbm.at[c, pl.ds(r0, CHUNK)], rows[0])
                pltpu.sync_copy(rows[0], tabs.at[pl.ds(r0, CHUNK)])
            tab = tabs
        else:
            tab = tab_hbm.at[c]

        def zrow(r, _):
            for j in range(Dh // 16):
                rows[0][r, pl.ds(j * 16, 16)] = jnp.zeros((16,), jnp.float32)
            return 0

        lax.fori_loop(0, CHUNK, zrow, 0)
        for j in range(NCOPY):
            pltpu.sync_copy(rows[0], acc.at[pl.ds(s * RPT + j * CHUNK, CHUNK)])
        pltpu.sync_copy(src_hbm.at[pl.ds(s * NITER_T, NITER_T)], srca)
        pltpu.sync_copy(dst_hbm.at[pl.ds(s * NITER_T, NITER_T)], dsta)
        plsc.subcore_barrier()

        for b in range(LEAD):
            pltpu.async_copy(tab.at[srca.at[b]], rows[b], gsem[b])

        def quad(q, _):
            for b in range(NBUF):
                g = NBUF * q + b
                bg = (b + LEAD) % NBUF
                g2 = g + LEAD

                @pl.when(jnp.logical_and(g2 >= NBUF, g2 < NITER_T))
                def _():
                    pltpu.make_async_copy(rows[bg], acc.at[dsta.at[g2 - NBUF]],
                                          ssem[bg]).wait()

                @pl.when(g2 < NITER_T)
                def _():
                    pltpu.async_copy(tab.at[srca.at[g2]], rows[bg], gsem[bg])

                pltpu.make_async_copy(tab.at[srca.at[g]], rows[b], gsem[b]).wait()
                pltpu.async_copy(rows[b], acc.at[dsta.at[g]], ssem[b], add=True)
            return 0

        lax.fori_loop(0, NITER_T // NBUF, quad, 0)
        for b in range(NBUF):
            g = NITER_T - NBUF + b
            pltpu.make_async_copy(rows[b], acc.at[dsta.at[g]], ssem[b]).wait()
        plsc.subcore_barrier()

        for j in range(NCOPY):
            r0 = s * RPT + j * CHUNK
            pltpu.sync_copy(acc.at[pl.ds(r0, CHUNK)], rows[0])
            pltpu.sync_copy(rows[0], out_hbm.at[c, pl.ds(r0, CHUNK)])

    return k(src3, dst3, tab_split)


RB = 2048
KB = 1024


def _img_body(img_ref, wm0_ref, bm0_ref, wm1_ref, bm1_ref, out_ref, acc_ref):
    kstep = pl.program_id(0)

    @pl.when(kstep == 0)
    def _():
        acc_ref[...] = jnp.zeros_like(acc_ref)

    acc_ref[...] += _dot(img_ref[...], wm0_ref[...])

    @pl.when(kstep == IMG // KB - 1)
    def _():
        y = acc_ref[...] + bm0_ref[...][None, :]
        out_ref[...] = _dot(y, wm1_ref[...]) + bm1_ref[...][None, :]


def _img_mlp(img, Wm0, bm0, Wm1, bm1):
    return pl.pallas_call(
        _img_body,
        grid=(IMG // KB,),
        in_specs=[
            pl.BlockSpec((G, KB), lambda k: (0, k)),
            pl.BlockSpec((KB, 1024), lambda k: (k, 0)),
            pl.BlockSpec((1024,), lambda k: (0,)),
            pl.BlockSpec((1024, HID), lambda k: (0, 0)),
            pl.BlockSpec((HID,), lambda k: (0,)),
        ],
        out_specs=pl.BlockSpec((G, HID), lambda k: (0, 0)),
        out_shape=jax.ShapeDtypeStruct((G, HID), jnp.float32),
        scratch_shapes=[pltpu.VMEM((G, 1024), jnp.float32)],
    )(img, Wm0, bm0, Wm1, bm1)


def _xw_body(x_ref, w1_ref, xw_ref):
    xw_ref[...] = _dot(x_ref[...], w1_ref[...])


def _xw(x_pad, W1):
    return pl.pallas_call(
        _xw_body,
        grid=(NPAD // RB,),
        in_specs=[
            pl.BlockSpec((RB, F_IN), lambda i: (i, 0)),
            pl.BlockSpec((F_IN, F_IN), lambda i: (0, 0)),
        ],
        out_specs=pl.BlockSpec((RB, F_IN), lambda i: (i, 0)),
        out_shape=jax.ShapeDtypeStruct((NPAD, F_IN), jnp.float32),
    )(x_pad, W1)


QW = F_IN // 4


def _prescale_body(deg_ref, xw_ref, dinv_ref, hs1a_ref, hs1b_ref):
    deg = deg_ref[0, :, 0] + deg_ref[1, :, 0] + 1.0
    dinv = lax.rsqrt(jnp.maximum(deg, 1.0))
    dinv_ref[...] = dinv
    hs = xw_ref[...] * dinv[:, None]
    hs1a_ref[0] = hs[:, 0 * QW:1 * QW]
    hs1a_ref[1] = hs[:, 1 * QW:2 * QW]
    hs1b_ref[0] = hs[:, 2 * QW:3 * QW]
    hs1b_ref[1] = hs[:, 3 * QW:4 * QW]


def _prescale(deg_part, xw):
    return pl.pallas_call(
        _prescale_body,
        grid=(NPAD // RB,),
        in_specs=[
            pl.BlockSpec((NCORES, RB, DW), lambda i: (0, i, 0)),
            pl.BlockSpec((RB, F_IN), lambda i: (i, 0)),
        ],
        out_specs=[
            pl.BlockSpec((RB,), lambda i: (i,)),
            pl.BlockSpec((NCORES, RB, QW), lambda i: (0, i, 0)),
            pl.BlockSpec((NCORES, RB, QW), lambda i: (0, i, 0)),
        ],
        out_shape=[
            jax.ShapeDtypeStruct((NPAD,), jnp.float32),
            jax.ShapeDtypeStruct((NCORES, NPAD, QW), jnp.float32),
            jax.ShapeDtypeStruct((NCORES, NPAD, QW), jnp.float32),
        ],
    )(deg_part, xw)


def _mid_body(agga_ref, aggb_ref, hs1a_ref, hs1b_ref, dinv_ref, b1_ref,
              w2_ref, hs2_ref):
    dinv = dinv_ref[...]
    tot = (jnp.concatenate([agga_ref[0], agga_ref[1],
                            aggb_ref[0], aggb_ref[1]], axis=1)
           + jnp.concatenate([hs1a_ref[0], hs1a_ref[1],
                              hs1b_ref[0], hs1b_ref[1]], axis=1))
    h1 = jnp.maximum(tot * dinv[:, None] + b1_ref[...][None, :], 0.0)
    hs2 = _dot(h1, w2_ref[...]) * dinv[:, None]
    hs2_ref[0] = hs2[:, :HID // 2]
    hs2_ref[1] = hs2[:, HID // 2:]


def _mid_layer(agg1a, agg1b, hs1a, hs1b, dinv, b1, W2):
    return pl.pallas_call(
        _mid_body,
        grid=(NPAD // RB,),
        in_specs=[
            pl.BlockSpec((NCORES, RB, QW), lambda i: (0, i, 0)),
            pl.BlockSpec((NCORES, RB, QW), lambda i: (0, i, 0)),
            pl.BlockSpec((NCORES, RB, QW), lambda i: (0, i, 0)),
            pl.BlockSpec((NCORES, RB, QW), lambda i: (0, i, 0)),
            pl.BlockSpec((RB,), lambda i: (i,)),
            pl.BlockSpec((F_IN,), lambda i: (0,)),
            pl.BlockSpec((F_IN, HID), lambda i: (0, 0)),
        ],
        out_specs=pl.BlockSpec((NCORES, RB, HID // 2), lambda i: (0, i, 0)),
        out_shape=jax.ShapeDtypeStruct((NCORES, NPAD, HID // 2), jnp.float32),
    )(agg1a, agg1b, hs1a, hs1b, dinv, b1, W2)


def _final_body(agg_ref, hs2_ref, dinv_ref, b2_ref, batch_ref, x0_ref,
                wmx_ref, bmx_ref, wfc_ref, bfc_ref, out_ref, pool_ref):
    i = pl.program_id(0)

    @pl.when(i == 0)
    def _():
        pool_ref[...] = jnp.zeros_like(pool_ref)

    dinv = dinv_ref[...]
    tot = (jnp.concatenate([agg_ref[0], agg_ref[1]], axis=1)
           + jnp.concatenate([hs2_ref[0], hs2_ref[1]], axis=1))
    h2 = jnp.maximum(tot * dinv[:, None] + b2_ref[...][None, :], 0.0)
    gid = lax.broadcasted_iota(jnp.int32, (G, RB), 0)
    seg = (batch_ref[...][None, :] == gid).astype(jnp.float32)
    pool_ref[...] += _dot(seg, h2)

    @pl.when(i == NPAD // RB - 1)
    def _():
        xg = _dot(pool_ref[...], wmx_ref[...]) + bmx_ref[...][None, :]
        xt = jnp.concatenate([x0_ref[...], xg], axis=1)
        logits = _dot(xt, wfc_ref[...]) + bfc_ref[...][None, :]
        m = jnp.max(logits, axis=1, keepdims=True)
        lse = m + jnp.log(jnp.sum(jnp.exp(logits - m), axis=1, keepdims=True))
        out_ref[...] = logits - lse


def _final(agg2, hs2, dinv, b2, batch_pad, x0, Wmx, bmx, Wfc, bfc):
    return pl.pallas_call(
        _final_body,
        grid=(NPAD // RB,),
        in_specs=[
            pl.BlockSpec((NCORES, RB, HID // 2), lambda i: (0, i, 0)),
            pl.BlockSpec((NCORES, RB, HID // 2), lambda i: (0, i, 0)),
            pl.BlockSpec((RB,), lambda i: (i,)),
            pl.BlockSpec((HID,), lambda i: (0,)),
            pl.BlockSpec((RB,), lambda i: (i,)),
            pl.BlockSpec((G, HID), lambda i: (0, 0)),
            pl.BlockSpec((HID, HID), lambda i: (0, 0)),
            pl.BlockSpec((HID,), lambda i: (0,)),
            pl.BlockSpec((2 * HID, NC), lambda i: (0, 0)),
            pl.BlockSpec((NC,), lambda i: (0,)),
        ],
        out_specs=pl.BlockSpec((G, NC), lambda i: (0, 0)),
        out_shape=jax.ShapeDtypeStruct((G, NC), jnp.float32),
        scratch_shapes=[pltpu.VMEM((G, HID), jnp.float32)],
    )(agg2, hs2, dinv, b2, batch_pad, x0, Wmx, bmx, Wfc, bfc)


def kernel(x, edge_index, img_features, batch, W1, b1, W2, b2,
           Wm0, bm0, Wm1, bm1, Wmx, bmx, Wfc, bfc):
    src = edge_index[0].astype(jnp.int32)
    dst = edge_index[1].astype(jnp.int32)
    npd = EPAD - E
    src_pad = jnp.concatenate([src, jnp.full((npd,), N, jnp.int32)])
    dst_pad = jnp.concatenate(
        [dst, N + (jnp.arange(npd, dtype=jnp.int32) % (NPAD - N))])
    src2 = src_pad.reshape(EPAD // CHUNK, CHUNK)
    dst2 = dst_pad.reshape(EPAD // CHUNK, CHUNK)
    x_pad = jnp.pad(x, ((0, NPAD - N), (0, 0)))
    batch_pad = jnp.concatenate(
        [batch.astype(jnp.int32), jnp.full((NPAD - N,), G, jnp.int32)])
    x0 = _img_mlp(img_features, Wm0, bm0, Wm1, bm1)

    xw = _xw(x_pad, W1)
    deg_part = _sc_hist(dst2)
    dinv, hs1a, hs1b = _prescale(deg_part, xw)
    agg1a = _sc_aggregate(src2, dst2, hs1a, QW, stage_tab=True)
    agg1b = _sc_aggregate(src2, dst2, hs1b, QW, stage_tab=True)
    hs2 = _mid_layer(agg1a, agg1b, hs1a, hs1b, dinv, b1, W2)
    hs2, x0 = lax.optimization_barrier((hs2, x0))
    agg2 = _sc_aggregate(src2, dst2, hs2, HID // 2, stage_tab=True)
    return _final(agg2, hs2, dinv, b2, batch_pad, x0, Wmx, bmx, Wfc, bfc)
